# baseline (speedup 1.0000x reference)
import functools

import jax
import jax.numpy as jnp
from jax import lax
from jax.experimental import pallas as pl
from jax.experimental.pallas import tpu as pltpu
from jax.experimental.pallas import tpu_sc as plsc

_B = 16384
_L = 50
_NH = 64
_NROWS = 1000008
_ZROW = 1000002
_NC = 2
_NS = 16
_NW = _NC * _NS
_RPW = _B // _NW
_G = 8
_NCHUNK = _RPW // _G
_CLEN = _G * _L
_NVEC = _CLEN // 16
_WLEN = _RPW * _L

_mesh = plsc.VectorSubcoreMesh(core_axis_name="c", subcore_axis_name="s")


@functools.partial(
    pl.kernel,
    out_type=jax.ShapeDtypeStruct((_B, _NH), jnp.bfloat16),
    mesh=_mesh,
    scratch_types=[
        pltpu.VMEM((_WLEN,), jnp.int32),
        pltpu.VMEM((_CLEN, _NH), jnp.bfloat16),
        pltpu.VMEM((_CLEN, _NH), jnp.bfloat16),
        pltpu.VMEM((_G, _NH), jnp.bfloat16),
        pltpu.SemaphoreType.DMA,
        pltpu.SemaphoreType.DMA,
    ],
    compiler_params=pltpu.CompilerParams(use_tc_tiling_on_sc=False),
)
def _pool(ctx_hbm, table_hbm, out_hbm, idx_v, rows0, rows1, out_v, sem0, sem1):
    wid = lax.axis_index("s") * _NC + lax.axis_index("c")
    base = wid * _RPW
    pltpu.sync_copy(ctx_hbm.at[pl.ds(base * _L, _WLEN)], idx_v)
    bufs = (rows0, rows1)
    sems = (sem0, sem1)

    def fire(g, buf, sem):
        off = g * _CLEN
        for k in range(_NVEC):
            vec = idx_v[pl.ds(off + k * 16, 16)]
            vec = jnp.where(vec == 0, jnp.int32(_ZROW), vec)
            pltpu.async_copy(
                table_hbm.at[vec], buf.at[pl.ds(k * 16, 16), :], sem
            )

    def drain(buf, sem):
        pltpu.make_async_copy(
            table_hbm.at[pl.ds(0, _CLEN), :], buf, sem
        ).wait()

    def compute(g, buf):
        def row_body(i, carry):
            r0 = i * _L
            accs = [jnp.full((32,), -jnp.inf, jnp.bfloat16) for _ in range(2)]
            for j in range(_L):
                for c2 in range(2):
                    vals = buf[r0 + j, pl.ds(c2 * 32, 32)]
                    accs[c2] = jnp.maximum(accs[c2], vals)
            for c2 in range(2):
                out_v[i, pl.ds(c2 * 32, 32)] = accs[c2]
            return carry

        lax.fori_loop(0, _G, row_body, 0)
        pltpu.sync_copy(out_v, out_hbm.at[pl.ds(base + g * _G, _G), :])

    fire(0, bufs[0], sems[0])

    def body(g2, carry):
        for b2 in range(2):
            g = g2 * 2 + b2
            drain(bufs[b2], sems[b2])

            @pl.when(g + 1 < _NCHUNK)
            def _next():
                fire(g + 1, bufs[1 - b2], sems[1 - b2])

            compute(g, bufs[b2])
        return carry

    lax.fori_loop(0, _NCHUNK // 2, body, 0)


def _fc_body(v_ref, w_ref, b_ref, o_ref):
    vf = v_ref[:, :].astype(jnp.float32)
    o_ref[:, :] = (
        lax.dot_general(
            vf, w_ref[:, :],
            dimension_numbers=(((1,), (1,)), ((), ())),
            preferred_element_type=jnp.float32,
        )
        + b_ref[:, :]
    )


_FC_BLK = 2048


def _fc(v, W, b):
    return pl.pallas_call(
        _fc_body,
        grid=(_B // _FC_BLK,),
        in_specs=[
            pl.BlockSpec((_FC_BLK, _NH), lambda i: (i, 0)),
            pl.BlockSpec((_NH, _NH), lambda i: (0, 0)),
            pl.BlockSpec((1, _NH), lambda i: (0, 0)),
        ],
        out_specs=pl.BlockSpec((_FC_BLK, _NH), lambda i: (i, 0)),
        out_shape=jax.ShapeDtypeStruct((_B, _NH), jnp.float32),
    )(v, W, b.reshape(1, _NH))


def kernel(context, table, W, b):
    tbf = jnp.concatenate(
        [table.astype(jnp.bfloat16),
         jnp.zeros((_NROWS - table.shape[0], _NH), jnp.bfloat16)],
        axis=0,
    )
    v = _pool(context.reshape(_B * _L), tbf)
    return _fc(v, W, b)
